# baseline (speedup 1.0000x reference)
import functools

import jax
import jax.numpy as jnp
from jax import lax
from jax.experimental import pallas as pl
from jax.experimental.pallas import tpu as pltpu
from jax.experimental.pallas import tpu_sc as plsc

K = 10
NUM_LEVELS = 2 ** K
BASE_SLICE = 2.0 ** (-K)
INV_SLICE = float(2.0 ** K)
N = 4194304

SC_N = N // 2
TC_N = N - SC_N

NUM_CORES = 2
NUM_SUBCORES = 16
NW = NUM_CORES * NUM_SUBCORES
PER_WORKER = SC_N // NW
CHUNK = 16384
NCHUNK = PER_WORKER // CHUNK
LANES = 16
GROUPS = CHUNK // LANES
UNROLL = 8

TC_COLS = 1024
ROWS = N // TC_COLS
TC_ROWS = TC_N // TC_COLS
TC_ROW0 = SC_N // TC_COLS
TC_BLOCK_ROWS = 256


def _quantize16(v):
    y = jnp.minimum(jnp.maximum(v * INV_SLICE, jnp.float32(0.0)),
                    jnp.float32(NUM_LEVELS - 1))
    return y.astype(jnp.int32).astype(jnp.float32) * jnp.float32(BASE_SLICE)


def _quantize_chunk(buf):
    def body(i, carry):
        base = i * (LANES * UNROLL)
        for j in range(UNROLL):
            sl = pl.ds(base + j * LANES, LANES)
            buf[sl] = _quantize16(buf[sl])
        return carry
    lax.fori_loop(0, GROUPS // UNROLL, body, 0)


@functools.partial(
    pl.kernel,
    mesh=plsc.VectorSubcoreMesh(core_axis_name="c", subcore_axis_name="s"),
    out_type=jax.ShapeDtypeStruct((N,), jnp.float32),
    scratch_types=[
        pltpu.VMEM((CHUNK,), jnp.float32),
        pltpu.VMEM((CHUNK,), jnp.float32),
        pltpu.SemaphoreType.DMA,
        pltpu.SemaphoreType.DMA,
        pltpu.SemaphoreType.DMA,
        pltpu.SemaphoreType.DMA,
    ],
)
def _sc_encode(x_hbm, out_hbm, buf0, buf1, si0, si1, so0, so1):
    wid = lax.axis_index("s") * NUM_CORES + lax.axis_index("c")
    base = wid * PER_WORKER
    bufs = (buf0, buf1)
    in_sems = (si0, si1)
    out_sems = (so0, so1)
    in_copies = [None, None]
    out_copies = [None, None]

    in_copies[0] = pltpu.async_copy(
        x_hbm.at[pl.ds(base, CHUNK)], bufs[0], in_sems[0])
    for k in range(NCHUNK):
        cur = k % 2
        nxt = (k + 1) % 2
        if k + 1 < NCHUNK:
            if out_copies[nxt] is not None:
                out_copies[nxt].wait()
            in_copies[nxt] = pltpu.async_copy(
                x_hbm.at[pl.ds(base + (k + 1) * CHUNK, CHUNK)],
                bufs[nxt], in_sems[nxt])
        in_copies[cur].wait()
        _quantize_chunk(bufs[cur])
        out_copies[cur] = pltpu.async_copy(
            bufs[cur], out_hbm.at[pl.ds(base + k * CHUNK, CHUNK)],
            out_sems[cur])
    out_copies[(NCHUNK - 2) % 2].wait()
    out_copies[(NCHUNK - 1) % 2].wait()


def _tc_body(x_ref, o_ref):
    o_ref[...] = _quantize16(x_ref[...])


def _tc_encode(xr):
    return pl.pallas_call(
        _tc_body,
        grid=(TC_ROWS // TC_BLOCK_ROWS,),
        in_specs=[pl.BlockSpec((TC_BLOCK_ROWS, TC_COLS),
                               lambda i: (TC_ROW0 // TC_BLOCK_ROWS + i, 0))],
        out_specs=pl.BlockSpec((TC_BLOCK_ROWS, TC_COLS), lambda i: (i, 0)),
        out_shape=jax.ShapeDtypeStruct((TC_ROWS, TC_COLS), jnp.float32),
    )(xr)


def kernel(x):
    out_full = _sc_encode(x)
    out_tc = _tc_encode(x.reshape(ROWS, TC_COLS))
    return lax.dynamic_update_slice(out_full, out_tc.reshape(-1), (SC_N,))


# speedup vs baseline: 1.7934x; 1.7934x over previous
import functools

import jax
import jax.numpy as jnp
from jax import lax
from jax.experimental import pallas as pl
from jax.experimental.pallas import tpu as pltpu
from jax.experimental.pallas import tpu_sc as plsc

K = 10
NUM_LEVELS = 2 ** K
BASE_SLICE = 2.0 ** (-K)
INV_SLICE = float(2.0 ** K)
N = 4194304

NUM_CORES = 2
NUM_SUBCORES = 16
NW = NUM_CORES * NUM_SUBCORES
PER_WORKER = N // NW
CHUNK = 16384
NCHUNK = PER_WORKER // CHUNK
LANES = 16
GROUPS = CHUNK // LANES
UNROLL = 8


def _quantize16(v):
    y = jnp.minimum(jnp.maximum(v * INV_SLICE, jnp.float32(0.0)),
                    jnp.float32(NUM_LEVELS - 1))
    return y.astype(jnp.int32).astype(jnp.float32) * jnp.float32(BASE_SLICE)


def _quantize_chunk(buf):
    def body(i, carry):
        base = i * (LANES * UNROLL)
        for j in range(UNROLL):
            sl = pl.ds(base + j * LANES, LANES)
            buf[sl] = _quantize16(buf[sl])
        return carry
    lax.fori_loop(0, GROUPS // UNROLL, body, 0)


@functools.partial(
    pl.kernel,
    mesh=plsc.VectorSubcoreMesh(core_axis_name="c", subcore_axis_name="s"),
    out_type=jax.ShapeDtypeStruct((N,), jnp.float32),
    scratch_types=[
        pltpu.VMEM((CHUNK,), jnp.float32),
        pltpu.VMEM((CHUNK,), jnp.float32),
        pltpu.SemaphoreType.DMA,
        pltpu.SemaphoreType.DMA,
        pltpu.SemaphoreType.DMA,
        pltpu.SemaphoreType.DMA,
    ],
)
def _sc_encode(x_hbm, out_hbm, buf0, buf1, si0, si1, so0, so1):
    wid = lax.axis_index("s") * NUM_CORES + lax.axis_index("c")
    base = wid * PER_WORKER
    bufs = (buf0, buf1)
    in_sems = (si0, si1)
    out_sems = (so0, so1)
    in_copies = [None, None]
    out_copies = [None, None]

    in_copies[0] = pltpu.async_copy(
        x_hbm.at[pl.ds(base, CHUNK)], bufs[0], in_sems[0])
    for k in range(NCHUNK):
        cur = k % 2
        nxt = (k + 1) % 2
        if k + 1 < NCHUNK:
            if out_copies[nxt] is not None:
                out_copies[nxt].wait()
            in_copies[nxt] = pltpu.async_copy(
                x_hbm.at[pl.ds(base + (k + 1) * CHUNK, CHUNK)],
                bufs[nxt], in_sems[nxt])
        in_copies[cur].wait()
        _quantize_chunk(bufs[cur])
        out_copies[cur] = pltpu.async_copy(
            bufs[cur], out_hbm.at[pl.ds(base + k * CHUNK, CHUNK)],
            out_sems[cur])
    out_copies[(NCHUNK - 2) % 2].wait()
    out_copies[(NCHUNK - 1) % 2].wait()


def kernel(x):
    return _sc_encode(x)


# speedup vs baseline: 1.9439x; 1.0839x over previous
import functools

import jax
import jax.numpy as jnp
from jax import lax
from jax.experimental import pallas as pl
from jax.experimental.pallas import tpu as pltpu
from jax.experimental.pallas import tpu_sc as plsc

K = 10
NUM_LEVELS = 2 ** K
BASE_SLICE = 2.0 ** (-K)
INV_SLICE = float(2.0 ** K)
N = 4194304

NUM_CORES = 2
NUM_SUBCORES = 16
NW = NUM_CORES * NUM_SUBCORES
PER_WORKER = N // NW
CHUNK = 16384
NCHUNK = PER_WORKER // CHUNK
LANES = 16
GROUPS = CHUNK // LANES
UNROLL = 8


def _quantize16(v):
    y = jnp.minimum(jnp.maximum(v * INV_SLICE, jnp.float32(0.0)),
                    jnp.float32(NUM_LEVELS - 1))
    return y.astype(jnp.int32).astype(jnp.float32) * jnp.float32(BASE_SLICE)


def _quantize_chunk(buf):
    def body(i, carry):
        base = i * (LANES * UNROLL)
        for j in range(UNROLL):
            sl = pl.ds(base + j * LANES, LANES)
            buf[sl] = _quantize16(buf[sl])
        return carry
    lax.fori_loop(0, GROUPS // UNROLL, body, 0)


@functools.partial(
    pl.kernel,
    mesh=plsc.VectorSubcoreMesh(core_axis_name="c", subcore_axis_name="s"),
    out_type=jax.ShapeDtypeStruct((N,), jnp.float32),
    scratch_types=[
        pltpu.VMEM((CHUNK,), jnp.float32),
        pltpu.VMEM((CHUNK,), jnp.float32),
        pltpu.VMEM((CHUNK,), jnp.float32),
        pltpu.VMEM((CHUNK,), jnp.float32),
        pltpu.SemaphoreType.DMA,
        pltpu.SemaphoreType.DMA,
        pltpu.SemaphoreType.DMA,
        pltpu.SemaphoreType.DMA,
        pltpu.SemaphoreType.DMA,
        pltpu.SemaphoreType.DMA,
        pltpu.SemaphoreType.DMA,
        pltpu.SemaphoreType.DMA,
    ],
)
def _sc_encode(x_hbm, out_hbm, b0, b1, b2, b3,
               si0, si1, si2, si3, so0, so1, so2, so3):
    wid = lax.axis_index("s") * NUM_CORES + lax.axis_index("c")
    base = wid * PER_WORKER
    NBUF = 4
    bufs = (b0, b1, b2, b3)
    in_sems = (si0, si1, si2, si3)
    out_sems = (so0, so1, so2, so3)
    in_copies = [None] * NBUF
    out_copies = [None] * NBUF

    for k in range(min(NBUF - 1, NCHUNK)):
        in_copies[k] = pltpu.async_copy(
            x_hbm.at[pl.ds(base + k * CHUNK, CHUNK)], bufs[k], in_sems[k])
    for k in range(NCHUNK):
        cur = k % NBUF
        in_copies[cur].wait()
        _quantize_chunk(bufs[cur])
        out_copies[cur] = pltpu.async_copy(
            bufs[cur], out_hbm.at[pl.ds(base + k * CHUNK, CHUNK)],
            out_sems[cur])
        pre = k + NBUF - 1
        if pre < NCHUNK:
            pb = pre % NBUF
            if out_copies[pb] is not None:
                out_copies[pb].wait()
            in_copies[pb] = pltpu.async_copy(
                x_hbm.at[pl.ds(base + pre * CHUNK, CHUNK)],
                bufs[pb], in_sems[pb])
    for b in range(NBUF):
        if out_copies[b] is not None and b != (NCHUNK - 1) % NBUF:
            out_copies[b].wait()
    out_copies[(NCHUNK - 1) % NBUF].wait()


def kernel(x):
    return _sc_encode(x)
